# initial kernel scaffold (unmeasured)
import jax
import jax.numpy as jnp
from jax import lax
from jax.experimental import pallas as pl
from jax.experimental.pallas import tpu as pltpu


def _exchange(logits):
    m, v = logits.shape

    def body(in_ref, out_ref, send_sem, recv_sem):
        my_x = lax.axis_index("x")
        my_y = lax.axis_index("y")
        nbr = (1 - my_x, my_y)

        barrier = pltpu.get_barrier_semaphore()
        pl.semaphore_signal(
            barrier, inc=1, device_id=nbr, device_id_type=pl.DeviceIdType.MESH
        )
        pl.semaphore_wait(barrier, 1)

        rdma = pltpu.make_async_remote_copy(
            src_ref=in_ref,
            dst_ref=out_ref,
            send_sem=send_sem,
            recv_sem=recv_sem,
            device_id=nbr,
            device_id_type=pl.DeviceIdType.MESH,
        )
        rdma.start()
        rdma.wait()

    return pl.pallas_call(
        body,
        out_shape=jax.ShapeDtypeStruct((m, v), logits.dtype),
        in_specs=[pl.BlockSpec(memory_space=pltpu.ANY)],
        out_specs=pl.BlockSpec(memory_space=pltpu.ANY),
        scratch_shapes=[
            pltpu.SemaphoreType.DMA,
            pltpu.SemaphoreType.DMA,
        ],
        compiler_params=pltpu.CompilerParams(collective_id=0),
    )(logits)


def kernel(x, W):
    logits = jnp.dot(x, W)
    other = _exchange(logits)
    my_x = lax.axis_index("x")
    full = lax.cond(
        my_x == 0,
        lambda: jnp.concatenate([logits, other], axis=1),
        lambda: jnp.concatenate([other, logits], axis=1),
    )
    return jax.nn.softmax(full, axis=-1)


# baseline (device time: 1142011 ns/iter reference)
import jax
import jax.numpy as jnp
from jax import lax
from jax.experimental import pallas as pl
from jax.experimental.pallas import tpu as pltpu


def _exchange(logits):
    m, v = logits.shape

    def body(in_ref, out_ref, send_sem, recv_sem):
        my_x = lax.axis_index("x")
        my_y = lax.axis_index("y")
        nbr = (1 - my_x, my_y)

        barrier = pltpu.get_barrier_semaphore()
        pl.semaphore_signal(
            barrier, inc=1, device_id=nbr, device_id_type=pl.DeviceIdType.MESH
        )
        pl.semaphore_wait(barrier, 1)

        rdma = pltpu.make_async_remote_copy(
            src_ref=in_ref,
            dst_ref=out_ref,
            send_sem=send_sem,
            recv_sem=recv_sem,
            device_id=nbr,
            device_id_type=pl.DeviceIdType.MESH,
        )
        rdma.start()
        rdma.wait()

    return pl.pallas_call(
        body,
        out_shape=jax.ShapeDtypeStruct((m, v), logits.dtype),
        in_specs=[pl.BlockSpec(memory_space=pl.ANY)],
        out_specs=pl.BlockSpec(memory_space=pl.ANY),
        scratch_shapes=[
            pltpu.SemaphoreType.DMA,
            pltpu.SemaphoreType.DMA,
        ],
        compiler_params=pltpu.CompilerParams(collective_id=0),
    )(logits)


def kernel(x, W):
    logits = jnp.dot(x, W)
    other = _exchange(logits)
    my_x = lax.axis_index("x")
    full = lax.cond(
        my_x == 0,
        lambda: jnp.concatenate([logits, other], axis=1),
        lambda: jnp.concatenate([other, logits], axis=1),
    )
    return jax.nn.softmax(full, axis=-1)


# device time: 899429 ns/iter; 1.2697x vs baseline; 1.2697x over previous
import jax
import jax.numpy as jnp
from jax import lax
from jax.experimental import pallas as pl
from jax.experimental.pallas import tpu as pltpu

M = 1024
K = 2048
V_HALF = 16384
CV = 1024
NC = V_HALF // CV


def _fused(x, W):
    def body(
        x_ref,
        w_ref,
        e_ref,
        s_ref,
        recv_buf,
        wv,
        lv,
        ov,
        rv,
        w_sem,
        st_sem,
        ld_sem,
        send_sems,
        recv_sems,
    ):
        my_x = lax.axis_index("x")
        my_y = lax.axis_index("y")
        nbr = (1 - my_x, my_y)
        my_base = my_x * V_HALF
        other_base = (1 - my_x) * V_HALF

        barrier = pltpu.get_barrier_semaphore()
        pl.semaphore_signal(
            barrier, inc=1, device_id=nbr, device_id_type=pl.DeviceIdType.MESH
        )
        pl.semaphore_wait(barrier, 1)

        xv = x_ref[...]

        w_cp = [None] * NC
        w_cp[0] = pltpu.make_async_copy(
            w_ref.at[:, pl.ds(0, CV)], wv.at[0], w_sem.at[0]
        )
        w_cp[0].start()

        sends = [None] * NC
        stores = []
        s_acc = jnp.zeros((M, 1), jnp.float32)

        def emit_chunk(e, col_base, j_st):
            nonlocal stores
            slot = j_st % 2
            if len(stores) >= 2:
                stores[-2].wait()
            ov[slot] = e
            st = pltpu.make_async_copy(
                ov.at[slot], e_ref.at[:, pl.ds(col_base, CV)], st_sem.at[slot]
            )
            st.start()
            stores.append(st)

        for j in range(NC):
            slot = j % 2
            if j + 1 < NC:
                w_cp[j + 1] = pltpu.make_async_copy(
                    w_ref.at[:, pl.ds((j + 1) * CV, CV)],
                    wv.at[(j + 1) % 2],
                    w_sem.at[(j + 1) % 2],
                )
                w_cp[j + 1].start()
            w_cp[j].wait()
            if j >= 2:
                sends[j - 2].wait_send()
            l = jnp.dot(xv, wv[slot], preferred_element_type=jnp.float32)
            lv[slot] = l
            sends[j] = pltpu.make_async_remote_copy(
                src_ref=lv.at[slot],
                dst_ref=recv_buf.at[:, pl.ds(j * CV, CV)],
                send_sem=send_sems.at[j],
                recv_sem=recv_sems.at[j],
                device_id=nbr,
                device_id_type=pl.DeviceIdType.MESH,
            )
            sends[j].start()
            e = jnp.exp(l)
            s_acc = s_acc + jnp.sum(e, axis=1, keepdims=True)
            emit_chunk(e, my_base + j * CV, j)

        sends[NC - 2].wait_send()
        sends[NC - 1].wait_send()

        for j in range(NC):
            recv = pltpu.make_async_remote_copy(
                src_ref=lv.at[0],
                dst_ref=recv_buf.at[:, pl.ds(j * CV, CV)],
                send_sem=send_sems.at[j],
                recv_sem=recv_sems.at[j],
                device_id=nbr,
                device_id_type=pl.DeviceIdType.MESH,
            )
            recv.wait_recv()
            ld = pltpu.make_async_copy(
                recv_buf.at[:, pl.ds(j * CV, CV)], rv, ld_sem
            )
            ld.start()
            ld.wait()
            e = jnp.exp(rv[...])
            s_acc = s_acc + jnp.sum(e, axis=1, keepdims=True)
            emit_chunk(e, other_base + j * CV, NC + j)

        stores[-2].wait()
        stores[-1].wait()
        s_ref[...] = s_acc

    out_shape = (
        jax.ShapeDtypeStruct((M, 2 * V_HALF), jnp.float32),
        jax.ShapeDtypeStruct((M, 1), jnp.float32),
        jax.ShapeDtypeStruct((M, V_HALF), jnp.float32),
    )
    return pl.pallas_call(
        body,
        out_shape=out_shape,
        in_specs=[
            pl.BlockSpec(memory_space=pltpu.MemorySpace.VMEM),
            pl.BlockSpec(memory_space=pl.ANY),
        ],
        out_specs=(
            pl.BlockSpec(memory_space=pl.ANY),
            pl.BlockSpec(memory_space=pltpu.MemorySpace.VMEM),
            pl.BlockSpec(memory_space=pl.ANY),
        ),
        scratch_shapes=[
            pltpu.VMEM((2, K, CV), jnp.float32),
            pltpu.VMEM((2, M, CV), jnp.float32),
            pltpu.VMEM((2, M, CV), jnp.float32),
            pltpu.VMEM((M, CV), jnp.float32),
            pltpu.SemaphoreType.DMA((2,)),
            pltpu.SemaphoreType.DMA((2,)),
            pltpu.SemaphoreType.DMA,
            pltpu.SemaphoreType.DMA((NC,)),
            pltpu.SemaphoreType.DMA((NC,)),
        ],
        compiler_params=pltpu.CompilerParams(
            collective_id=0, vmem_limit_bytes=60 * 1024 * 1024
        ),
    )(x, W)


def kernel(x, W):
    e, s, _ = _fused(x, W)
    return e / s


# device time: 542866 ns/iter; 2.1037x vs baseline; 1.6568x over previous
import jax
import jax.numpy as jnp
from jax import lax
from jax.experimental import pallas as pl
from jax.experimental.pallas import tpu as pltpu

M = 1024
K = 2048
V_HALF = 16384
CV = 1024
NC = V_HALF // CV
HM = M // 2
FWD_LAG = 2


def _fused(x, W):
    def body(
        x_ref,
        w_ref,
        e_ref,
        s_ref,
        recv_buf,
        wv,
        lv,
        ov,
        rv,
        w_sem,
        st_sem,
        ld_sem,
        send_sems,
        recv_x_sems,
        fwd_sems,
        recv_y_sems,
    ):
        my_x = lax.axis_index("x")
        my_y = lax.axis_index("y")
        xnbr = (1 - my_x, my_y)
        ynbr = (my_x, 1 - my_y)
        my_base = my_x * V_HALF
        other_base = (1 - my_x) * V_HALF
        row_base = my_y * HM
        other_row = (1 - my_y) * HM

        barrier = pltpu.get_barrier_semaphore()
        for tgt in (xnbr, ynbr):
            pl.semaphore_signal(
                barrier, inc=1, device_id=tgt, device_id_type=pl.DeviceIdType.MESH
            )
        pl.semaphore_wait(barrier, 2)

        xv = x_ref[...]

        w_cp = [None] * NC
        w_cp[0] = pltpu.make_async_copy(
            w_ref.at[:, pl.ds(0, CV)], wv.at[0], w_sem.at[0]
        )
        w_cp[0].start()

        sends = [None] * NC
        fwds = [None] * NC
        stores = []
        s_acc = jnp.zeros((M, 1), jnp.float32)

        def emit_chunk(e, col_base, j_st):
            slot = j_st % 2
            if len(stores) >= 2:
                stores[-2].wait()
            ov[slot] = e
            st = pltpu.make_async_copy(
                ov.at[slot], e_ref.at[:, pl.ds(col_base, CV)], st_sem.at[slot]
            )
            st.start()
            stores.append(st)

        def wait_recv_x(j):
            pltpu.make_async_remote_copy(
                src_ref=lv.at[0, pl.ds(0, HM)],
                dst_ref=recv_buf.at[pl.ds(row_base, HM), pl.ds(j * CV, CV)],
                send_sem=send_sems.at[j],
                recv_sem=recv_x_sems.at[j],
                device_id=xnbr,
                device_id_type=pl.DeviceIdType.MESH,
            ).wait_recv()

        def start_fwd(j):
            fwds[j] = pltpu.make_async_remote_copy(
                src_ref=recv_buf.at[pl.ds(row_base, HM), pl.ds(j * CV, CV)],
                dst_ref=recv_buf.at[pl.ds(row_base, HM), pl.ds(j * CV, CV)],
                send_sem=fwd_sems.at[j],
                recv_sem=recv_y_sems.at[j],
                device_id=ynbr,
                device_id_type=pl.DeviceIdType.MESH,
            )
            fwds[j].start()

        for j in range(NC):
            slot = j % 2
            if j + 1 < NC:
                w_cp[j + 1] = pltpu.make_async_copy(
                    w_ref.at[:, pl.ds((j + 1) * CV, CV)],
                    wv.at[(j + 1) % 2],
                    w_sem.at[(j + 1) % 2],
                )
                w_cp[j + 1].start()
            w_cp[j].wait()
            if j >= 2:
                sends[j - 2].wait_send()
            l = jnp.dot(xv, wv[slot], preferred_element_type=jnp.float32)
            lv[slot] = l
            sends[j] = pltpu.make_async_remote_copy(
                src_ref=lv.at[slot, pl.ds(row_base, HM)],
                dst_ref=recv_buf.at[pl.ds(row_base, HM), pl.ds(j * CV, CV)],
                send_sem=send_sems.at[j],
                recv_sem=recv_x_sems.at[j],
                device_id=xnbr,
                device_id_type=pl.DeviceIdType.MESH,
            )
            sends[j].start()
            if j >= FWD_LAG:
                wait_recv_x(j - FWD_LAG)
                start_fwd(j - FWD_LAG)
            e = jnp.exp(l)
            s_acc = s_acc + jnp.sum(e, axis=1, keepdims=True)
            emit_chunk(e, my_base + j * CV, j)

        sends[NC - 2].wait_send()
        sends[NC - 1].wait_send()
        for j in range(NC - FWD_LAG, NC):
            wait_recv_x(j)
            start_fwd(j)

        for j in range(NC):
            pltpu.make_async_remote_copy(
                src_ref=lv.at[0, pl.ds(0, HM)],
                dst_ref=recv_buf.at[pl.ds(other_row, HM), pl.ds(j * CV, CV)],
                send_sem=send_sems.at[j],
                recv_sem=recv_y_sems.at[j],
                device_id=ynbr,
                device_id_type=pl.DeviceIdType.MESH,
            ).wait_recv()
            ld = pltpu.make_async_copy(
                recv_buf.at[:, pl.ds(j * CV, CV)], rv, ld_sem
            )
            ld.start()
            ld.wait()
            e = jnp.exp(rv[...])
            s_acc = s_acc + jnp.sum(e, axis=1, keepdims=True)
            emit_chunk(e, other_base + j * CV, NC + j)

        for j in range(NC):
            fwds[j].wait_send()
        stores[-2].wait()
        stores[-1].wait()
        s_ref[...] = s_acc

    out_shape = (
        jax.ShapeDtypeStruct((M, 2 * V_HALF), jnp.float32),
        jax.ShapeDtypeStruct((M, 1), jnp.float32),
        jax.ShapeDtypeStruct((M, V_HALF), jnp.float32),
    )
    return pl.pallas_call(
        body,
        out_shape=out_shape,
        in_specs=[
            pl.BlockSpec(memory_space=pltpu.MemorySpace.VMEM),
            pl.BlockSpec(memory_space=pl.ANY),
        ],
        out_specs=(
            pl.BlockSpec(memory_space=pl.ANY),
            pl.BlockSpec(memory_space=pltpu.MemorySpace.VMEM),
            pl.BlockSpec(memory_space=pl.ANY),
        ),
        scratch_shapes=[
            pltpu.VMEM((2, K, CV), jnp.float32),
            pltpu.VMEM((2, M, CV), jnp.float32),
            pltpu.VMEM((2, M, CV), jnp.float32),
            pltpu.VMEM((M, CV), jnp.float32),
            pltpu.SemaphoreType.DMA((2,)),
            pltpu.SemaphoreType.DMA((2,)),
            pltpu.SemaphoreType.DMA,
            pltpu.SemaphoreType.DMA((NC,)),
            pltpu.SemaphoreType.DMA((NC,)),
            pltpu.SemaphoreType.DMA((NC,)),
            pltpu.SemaphoreType.DMA((NC,)),
        ],
        compiler_params=pltpu.CompilerParams(
            collective_id=0, vmem_limit_bytes=60 * 1024 * 1024
        ),
    )(x, W)


def kernel(x, W):
    e, s, _ = _fused(x, W)
    return e / s
